# baseline (speedup 1.0000x reference)
import functools

import numpy as np
import jax
import jax.numpy as jnp
from jax.experimental import pallas as pl

N = 10000
E = 320000
F_IN = 128
NHID = 128
NCLS = 10
K = 3
NLAYERS = 3
NGRAPH = 64
RATIO = 0.5


def _propagate(x, src, dst, ew, n):
    deg_out = jax.ops.segment_sum(ew, src, num_segments=n)
    deg_in = jax.ops.segment_sum(ew, dst, num_segments=n)
    norm = ew / jnp.sqrt(jnp.clip(deg_out[src], 1.0) * jnp.clip(deg_in[dst], 1.0))
    return jax.ops.segment_sum(x[src] * norm[:, None], dst, num_segments=n)


def _licheb(x, src, dst, ew, nmask, W, b, p):
    n = x.shape[0]
    m = nmask.astype(x.dtype)[:, None]
    Tx0 = x * m
    out = Tx0 @ W[0]
    Tx1 = -_propagate(Tx0, src, dst, ew, n)
    out = out + Tx1 @ W[1]
    for k in range(2, W.shape[0]):
        Tx2 = -2.0 * _propagate(Tx1, src, dst, ew, n) - Tx0
        out = out + Tx2 @ W[k]
        Tx0, Tx1 = Tx1, Tx2
    out = out + b
    score = jnp.tanh(out @ p)
    return out, score


def _readout(x, batch, nmask, ngraph):
    m = nmask.astype(x.dtype)[:, None]
    xm = jnp.where(nmask[:, None], x, -1e30)
    gmax = jax.ops.segment_max(xm, batch, num_segments=ngraph)
    s = jax.ops.segment_sum(x * m, batch, num_segments=ngraph)
    cnt = jax.ops.segment_sum(nmask.astype(x.dtype), batch, num_segments=ngraph)
    gmean = s / jnp.clip(cnt, 1.0)[:, None]
    return jnp.concatenate([gmax, gmean], axis=1)


def _mlp_kernel(xo_ref, lw1_ref, lb1_ref, lw2_ref, lb2_ref, lw3_ref, lb3_ref, o_ref):
    h = xo_ref[...] @ lw1_ref[...] + lb1_ref[...][None, :]
    h = jnp.maximum(h, 0.0)
    h = h @ lw2_ref[...] + lb2_ref[...][None, :]
    h = jnp.maximum(h, 0.0)
    logits = h @ lw3_ref[...] + lb3_ref[...][None, :]
    mx = jnp.max(logits, axis=-1, keepdims=True)
    z = logits - mx
    lse = jnp.log(jnp.sum(jnp.exp(z), axis=-1, keepdims=True))
    o_ref[...] = z - lse


@functools.partial(jax.jit)
def _mlp_head(x_out, lw1, lb1, lw2, lb2, lw3, lb3):
    return pl.pallas_call(
        _mlp_kernel,
        out_shape=jax.ShapeDtypeStruct((NGRAPH, NCLS), jnp.float32),
    )(x_out, lw1, lb1, lw2, lb2, lw3, lb3)


def kernel(x, edge_index, batch, W1, b1, p1, W2, b2, p2, W3, b3, p3, lw1, lb1, lw2, lb2, lw3, lb3):
    src = edge_index[0]
    dst = edge_index[1]
    n = x.shape[0]
    nmask = jnp.ones((n,), dtype=bool)
    ew = jnp.ones((src.shape[0],), dtype=x.dtype)
    convs = [(W1, b1, p1), (W2, b2, p2), (W3, b3, p3)]
    k_cur = n
    x_out = jnp.zeros((NGRAPH, 2 * NHID), dtype=x.dtype)
    for idx in range(NLAYERS - 1):
        W, b, p = convs[idx]
        x, score = _licheb(x, src, dst, ew, nmask, W, b, p)
        x = jax.nn.relu(x)
        k_cur = int(np.ceil(RATIO * k_cur))
        mscore = jnp.where(nmask, score, -jnp.inf)
        order = jnp.argsort(-mscore)
        keep = jnp.zeros((n,), dtype=bool).at[order[:k_cur]].set(True) & nmask
        x = x * (score * keep.astype(x.dtype))[:, None]
        nmask = keep
        ew = ew * keep[src].astype(x.dtype) * keep[dst].astype(x.dtype)
        x_out = x_out + jax.nn.relu(_readout(x, batch, nmask, NGRAPH))
    W, b, p = convs[NLAYERS - 1]
    x, score = _licheb(x, src, dst, ew, nmask, W, b, p)
    x = jax.nn.relu(x)
    x_out = x_out + jax.nn.relu(_readout(x, batch, nmask, NGRAPH))
    return _mlp_head(x_out, lw1, lb1, lw2, lb2, lw3, lb3)


# speedup vs baseline: 1.9125x; 1.9125x over previous
import functools

import numpy as np
import jax
import jax.numpy as jnp
from jax import lax
from jax.experimental import pallas as pl
from jax.experimental.pallas import tpu as pltpu, tpu_sc as plsc

N = 10000
E = 320000
F_IN = 128
NHID = 128
NCLS = 10
K = 3
NLAYERS = 3
NGRAPH = 64
RATIO = 0.5

NCORES = 2
NSUB = 16
NW = NCORES * NSUB
CHUNK = 128
CPW = 79
EPW = CPW * CHUNK
EPAD = NW * EPW
NA = 10112
ROWS_PT = NA // NSUB
WBC = (128, 128, 128, 128, 120)


@functools.partial(
    pl.kernel,
    out_type=jax.ShapeDtypeStruct((NCORES, NA, NHID), jnp.float32),
    mesh=plsc.VectorSubcoreMesh(
        core_axis_name="c", subcore_axis_name="s",
        num_cores=NCORES, num_subcores=NSUB),
    scratch_types=[
        pltpu.VMEM((CHUNK,), jnp.int32),
        pltpu.VMEM((CHUNK,), jnp.int32),
        pltpu.VMEM((CHUNK, NHID), jnp.float32),
        pltpu.VMEM_SHARED((NA, NHID), jnp.float32),
        pltpu.SemaphoreType.DMA,
    ],
)
def _sc_propagate(xs_hbm, src_hbm, dst_hbm, out_hbm, idx_s, idx_d, rows, acc, sem):
    c = lax.axis_index("c")
    s = lax.axis_index("s")
    wid = s * NCORES + c
    base = wid * EPW

    def zrow(i, carry):
        for j in range(NHID // 16):
            rows[i, pl.ds(j * 16, 16)] = jnp.zeros((16,), jnp.float32)
        return carry

    lax.fori_loop(0, CHUNK, zrow, 0)
    off_wb = 0
    for w in WBC:
        pltpu.sync_copy(rows.at[pl.ds(0, w)],
                        acc.at[pl.ds(s * ROWS_PT + off_wb, w)])
        off_wb += w
    plsc.subcore_barrier()

    def step(ci, carry):
        off = base + ci * CHUNK
        pltpu.sync_copy(src_hbm.at[pl.ds(off, CHUNK)], idx_s)
        pltpu.sync_copy(dst_hbm.at[pl.ds(off, CHUNK)], idx_d)
        pltpu.async_copy(xs_hbm.at[idx_s], rows, sem).wait()
        pltpu.sync_copy(rows, acc.at[idx_d], add=True)
        return carry

    lax.fori_loop(0, CPW, step, 0)
    plsc.subcore_barrier()

    off_wb = 0
    for w in WBC:
        r0 = s * ROWS_PT + off_wb
        pltpu.sync_copy(acc.at[pl.ds(r0, w)], out_hbm.at[c, pl.ds(r0, w)])
        off_wb += w


def _propagate(xpre, srcp, dst_eff):
    parts = _sc_propagate(xpre, srcp, dst_eff)
    return parts[0, :N] + parts[1, :N]


def _licheb(x, srcp, dst_eff, isq_out, isq_in, nmask, W, b, p):
    m = nmask.astype(x.dtype)[:, None]
    Tx0 = x * m
    out = Tx0 @ W[0]
    Tx1 = -(isq_in[:, None] * _propagate(Tx0 * isq_out[:, None], srcp, dst_eff))
    out = out + Tx1 @ W[1]
    for k in range(2, W.shape[0]):
        Tx2 = (-2.0 * isq_in[:, None]
               * _propagate(Tx1 * isq_out[:, None], srcp, dst_eff)) - Tx0
        out = out + Tx2 @ W[k]
        Tx0, Tx1 = Tx1, Tx2
    out = out + b
    score = jnp.tanh(out @ p)
    return out, score


def _readout(x, batch, nmask, ngraph):
    m = nmask.astype(x.dtype)[:, None]
    xm = jnp.where(nmask[:, None], x, -1e30)
    gmax = jax.ops.segment_max(xm, batch, num_segments=ngraph)
    s = jax.ops.segment_sum(x * m, batch, num_segments=ngraph)
    cnt = jax.ops.segment_sum(nmask.astype(x.dtype), batch, num_segments=ngraph)
    gmean = s / jnp.clip(cnt, 1.0)[:, None]
    return jnp.concatenate([gmax, gmean], axis=1)


def _mlp_kernel(xo_ref, lw1_ref, lb1_ref, lw2_ref, lb2_ref, lw3_ref, lb3_ref, o_ref):
    h = xo_ref[...] @ lw1_ref[...] + lb1_ref[...][None, :]
    h = jnp.maximum(h, 0.0)
    h = h @ lw2_ref[...] + lb2_ref[...][None, :]
    h = jnp.maximum(h, 0.0)
    logits = h @ lw3_ref[...] + lb3_ref[...][None, :]
    mx = jnp.max(logits, axis=-1, keepdims=True)
    z = logits - mx
    lse = jnp.log(jnp.sum(jnp.exp(z), axis=-1, keepdims=True))
    o_ref[...] = z - lse


def _mlp_head(x_out, lw1, lb1, lw2, lb2, lw3, lb3):
    return pl.pallas_call(
        _mlp_kernel,
        out_shape=jax.ShapeDtypeStruct((NGRAPH, NCLS), jnp.float32),
    )(x_out, lw1, lb1, lw2, lb2, lw3, lb3)


def kernel(x, edge_index, batch, W1, b1, p1, W2, b2, p2, W3, b3, p3,
           lw1, lb1, lw2, lb2, lw3, lb3):
    src = edge_index[0]
    dst = edge_index[1]
    n = x.shape[0]
    pad = EPAD - E
    srcp = jnp.concatenate([src, jnp.zeros((pad,), jnp.int32)])
    dstp = jnp.concatenate([dst, jnp.full((pad,), N, jnp.int32)])
    keep_e = jnp.concatenate([jnp.ones((E,), bool), jnp.zeros((pad,), bool)])

    nmask = jnp.ones((n,), dtype=bool)
    convs = [(W1, b1, p1), (W2, b2, p2), (W3, b3, p3)]
    k_cur = n
    x_out = jnp.zeros((NGRAPH, 2 * NHID), dtype=x.dtype)
    for idx in range(NLAYERS):
        W, b, p = convs[idx]
        ewf = keep_e[:E].astype(x.dtype)
        deg_out = jax.ops.segment_sum(ewf, src, num_segments=n)
        deg_in = jax.ops.segment_sum(ewf, dst, num_segments=n)
        isq_out = lax.rsqrt(jnp.clip(deg_out, 1.0))
        isq_in = lax.rsqrt(jnp.clip(deg_in, 1.0))
        dst_eff = jnp.where(keep_e, dstp, N)
        x, score = _licheb(x, srcp, dst_eff, isq_out, isq_in, nmask, W, b, p)
        x = jax.nn.relu(x)
        if idx == NLAYERS - 1:
            x_out = x_out + jax.nn.relu(_readout(x, batch, nmask, NGRAPH))
            break
        k_cur = int(np.ceil(RATIO * k_cur))
        mscore = jnp.where(nmask, score, -jnp.inf)
        order = jnp.argsort(-mscore)
        keep = jnp.zeros((n,), dtype=bool).at[order[:k_cur]].set(True) & nmask
        x = x * (score * keep.astype(x.dtype))[:, None]
        nmask = keep
        keep_e = keep_e & keep[srcp] & keep[dstp]
        x_out = x_out + jax.nn.relu(_readout(x, batch, nmask, NGRAPH))
    return _mlp_head(x_out, lw1, lb1, lw2, lb2, lw3, lb3)
